# baseline (speedup 1.0000x reference)
import functools

import jax
import jax.numpy as jnp
from jax import lax
from jax.experimental import pallas as pl
from jax.experimental.pallas import tpu as pltpu
from jax.experimental.pallas import tpu_sc as plsc

EPS = 1e-4
N_NODES = 10000
D = 128

NC, NS = 2, 16
NW = NC * NS
CHUNK = 128
AGG_ROWS = 10240
ROWS_PER_TILE = AGG_ROWS // NS


def _seg_sum_sc(x, idx_pairs, zeros_hbm, n_loop_chunks):
    mesh = plsc.VectorSubcoreMesh(core_axis_name="c", subcore_axis_name="s")

    @functools.partial(
        pl.kernel,
        out_type=jax.ShapeDtypeStruct((NC, AGG_ROWS, D), jnp.float32),
        mesh=mesh,
        scratch_types=[
            pltpu.VMEM((2, 2, CHUNK), jnp.int32),
            pltpu.VMEM((2, CHUNK, D), jnp.float32),
            pltpu.VMEM_SHARED((AGG_ROWS, D), jnp.float32),
            pltpu.SemaphoreType.DMA,
            pltpu.SemaphoreType.DMA,
        ],
    )
    def seg_kernel(x_hbm, idx_hbm, zero_hbm, out_hbm,
                   idx_v, rows_v, agg_sh, sem0, sem1):
        c = lax.axis_index("c")
        s = lax.axis_index("s")
        wid = c * NS + s
        sems = (sem0, sem1)
        pltpu.sync_copy(zero_hbm,
                        agg_sh.at[pl.ds(s * ROWS_PER_TILE, ROWS_PER_TILE)])
        plsc.subcore_barrier()

        for b in range(2):
            pltpu.sync_copy(idx_hbm.at[wid, b], idx_v.at[b])
            pltpu.async_copy(x_hbm.at[idx_v.at[b, 0]], rows_v.at[b],
                             sems[b])

        def body(p, carry):
            for b in range(2):
                j = 2 * p + b
                pltpu.make_async_copy(x_hbm.at[idx_v.at[b, 0]],
                                      rows_v.at[b], sems[b]).wait()
                pltpu.sync_copy(rows_v.at[b], agg_sh.at[idx_v.at[b, 1]],
                                add=True)
                pltpu.sync_copy(idx_hbm.at[wid, j + 2], idx_v.at[b])
                pltpu.async_copy(x_hbm.at[idx_v.at[b, 0]],
                                 rows_v.at[b], sems[b])
            return carry

        lax.fori_loop(0, n_loop_chunks // 2, body, 0)
        for b in range(2):
            pltpu.make_async_copy(x_hbm.at[idx_v.at[b, 0]],
                                  rows_v.at[b], sems[b]).wait()
        plsc.subcore_barrier()
        pltpu.sync_copy(agg_sh.at[pl.ds(s * ROWS_PER_TILE, ROWS_PER_TILE)],
                        out_hbm.at[c, pl.ds(s * ROWS_PER_TILE, ROWS_PER_TILE)])

    return seg_kernel(x, idx_pairs, zeros_hbm)


def _softplus(x):
    return jnp.maximum(x, 0.0) + jnp.log1p(jnp.exp(-jnp.abs(x)))


def _gelu(x):
    return 0.5 * x * (1.0 + lax.erf(x * 0.7071067811865476))


def _ln(x, g, b):
    m = jnp.mean(x, axis=-1, keepdims=True)
    v = jnp.mean((x - m) * (x - m), axis=-1, keepdims=True)
    return (x - m) * lax.rsqrt(v + 1e-5) * g + b


def _matT(x, w):
    return lax.dot_general(x, w, (((1,), (1,)), ((), ())),
                           preferred_element_type=jnp.float32)


_BLK = 1000


def _prelude_body(x_ref, wr_ref, br_ref, w1_ref, b1_ref, w2_ref, b2_ref,
                  grb_ref, brb_ref, gn_ref, bn_ref,
                  rate_ref, gamma_ref, xres_ref):
    x = x_ref[...]
    xres_ref[...] = _ln(x, gn_ref[...], bn_ref[...])
    rate_ref[...] = _softplus(_matT(x, wr_ref[...]) + br_ref[...])
    t = _softplus(_matT(x, w1_ref[...]) + b1_ref[...])
    gamma_ref[...] = _ln(_matT(t, w2_ref[...]) + b2_ref[...],
                         grb_ref[...], brb_ref[...])


def _prelude_tc(x, wr, br, w1, b1, w2, b2, grb, brb, gn, bn):
    n = x.shape[0]
    grid = (n // _BLK,)
    row_spec = pl.BlockSpec((_BLK, D), lambda i: (i, 0))
    w_spec = pl.BlockSpec((D, D), lambda i: (0, 0))
    v_spec = pl.BlockSpec((1, D), lambda i: (0, 0))
    shp = jax.ShapeDtypeStruct((n, D), jnp.float32)
    return pl.pallas_call(
        _prelude_body,
        grid=grid,
        in_specs=[row_spec,
                  w_spec, v_spec, w_spec, v_spec, w_spec, v_spec,
                  v_spec, v_spec, v_spec, v_spec],
        out_specs=[row_spec, row_spec, row_spec],
        out_shape=[shp, shp, shp],
    )(x, wr, br, w1, b1, w2, b2, grb, brb, gn, bn)


def _post_body(rate_ref, gamma_ref, xres_ref, a0_ref, a1_ref, deg_ref,
               wf1_ref, bf1_ref, wf2_ref, bf2_ref, out_ref):
    rate = rate_ref[...]
    agg = a0_ref[...] + a1_ref[...]
    h = (rate * agg + gamma_ref[...]) / (1.0 + rate * deg_ref[...] + EPS)
    u = _gelu(_matT(h, wf1_ref[...]) + bf1_ref[...])
    out_ref[...] = _matT(u, wf2_ref[...]) + bf2_ref[...] + xres_ref[...]


def _post_tc(rate, gamma, xres, agg0, agg1, deg2d, wf1, bf1, wf2, bf2):
    n = rate.shape[0]
    grid = (n // _BLK,)
    row_spec = pl.BlockSpec((_BLK, D), lambda i: (i, 0))
    deg_spec = pl.BlockSpec((_BLK, 1), lambda i: (i, 0))
    w_spec = pl.BlockSpec((D, D), lambda i: (0, 0))
    v_spec = pl.BlockSpec((1, D), lambda i: (0, 0))
    return pl.pallas_call(
        _post_body,
        grid=grid,
        in_specs=[row_spec, row_spec, row_spec, row_spec, row_spec,
                  deg_spec, w_spec, v_spec, w_spec, v_spec],
        out_specs=row_spec,
        out_shape=jax.ShapeDtypeStruct((n, D), jnp.float32),
    )(rate, gamma, xres, agg0, agg1, deg2d, wf1, bf1, wf2, bf2)


def kernel(x, edge_index, degree, W_rate, b_rate, W_rb1, b_rb1, W_rb2, b_rb2,
           g_rb, beta_rb, W_fc1, b_fc1, W_fc2, b_fc2, g_norm, beta_norm):
    e = edge_index.shape[1]
    quantum = NW * CHUNK
    n_loop_chunks = ((e + quantum - 1) // quantum + 1) // 2 * 2
    n_arr_chunks = n_loop_chunks + 2
    e_pad = n_arr_chunks * quantum
    pad = e_pad - e
    src_p = jnp.concatenate([edge_index[0], jnp.zeros((pad,), jnp.int32)])
    dst_p = jnp.concatenate(
        [edge_index[1], jnp.full((pad,), N_NODES, jnp.int32)])
    src_r = src_p.reshape(n_arr_chunks, NW, CHUNK).transpose(1, 0, 2)
    dst_r = dst_p.reshape(n_arr_chunks, NW, CHUNK).transpose(1, 0, 2)
    idx_pairs = jnp.stack([src_r, dst_r], axis=2)
    zeros_hbm = jnp.zeros((ROWS_PER_TILE, D), jnp.float32)

    agg = _seg_sum_sc(x, idx_pairs, zeros_hbm, n_loop_chunks)

    deg2d = degree[:, None]
    vec = lambda a: a.reshape(1, D)
    rate, gamma, xres = _prelude_tc(
        x, W_rate, vec(b_rate), W_rb1, vec(b_rb1), W_rb2, vec(b_rb2),
        vec(g_rb), vec(beta_rb), vec(g_norm), vec(beta_norm))
    out = _post_tc(rate, gamma, xres, agg[0, :N_NODES], agg[1, :N_NODES],
                   deg2d, W_fc1, vec(b_fc1), W_fc2, vec(b_fc2))
    return out


# speedup vs baseline: 1.4697x; 1.4697x over previous
import functools

import jax
import jax.numpy as jnp
from jax import lax
from jax.experimental import pallas as pl
from jax.experimental.pallas import tpu as pltpu
from jax.experimental.pallas import tpu_sc as plsc

EPS = 1e-4
N_NODES = 10000
D = 128

NC, NS = 2, 16
NW = NC * NS
CHUNK = 256
AGG_ROWS = 10240
ROWS_PER_TILE = AGG_ROWS // NS


def _seg_sum_sc(x, idx_pairs, zeros_hbm):
    n_chunks = idx_pairs.shape[1]
    mesh = plsc.VectorSubcoreMesh(core_axis_name="c", subcore_axis_name="s")

    @functools.partial(
        pl.kernel,
        out_type=jax.ShapeDtypeStruct((NC, AGG_ROWS, D), jnp.float32),
        mesh=mesh,
        scratch_types=[
            pltpu.VMEM((CHUNK,), jnp.int32),
            pltpu.VMEM((CHUNK,), jnp.int32),
            pltpu.VMEM((CHUNK, D), jnp.float32),
            pltpu.VMEM_SHARED((AGG_ROWS, D), jnp.float32),
            pltpu.SemaphoreType.DMA,
        ],
    )
    def seg_kernel(x_hbm, idx_hbm, zero_hbm, out_hbm,
                   src_v, dst_v, rows_v, agg_sh, sem):
        c = lax.axis_index("c")
        s = lax.axis_index("s")
        wid = c * NS + s
        pltpu.sync_copy(zero_hbm,
                        agg_sh.at[pl.ds(s * ROWS_PER_TILE, ROWS_PER_TILE)])
        plsc.subcore_barrier()

        def body(j, carry):
            pltpu.sync_copy(idx_hbm.at[wid, j, 0], src_v)
            pltpu.sync_copy(idx_hbm.at[wid, j, 1], dst_v)
            pltpu.async_copy(x_hbm.at[src_v], rows_v, sem).wait()
            pltpu.sync_copy(rows_v, agg_sh.at[dst_v], add=True)
            return carry

        lax.fori_loop(0, n_chunks, body, 0)
        plsc.subcore_barrier()
        pltpu.sync_copy(agg_sh.at[pl.ds(s * ROWS_PER_TILE, ROWS_PER_TILE)],
                        out_hbm.at[c, pl.ds(s * ROWS_PER_TILE, ROWS_PER_TILE)])

    return seg_kernel(x, idx_pairs, zeros_hbm)


def _softplus(x):
    return jnp.maximum(x, 0.0) + jnp.log1p(jnp.exp(-jnp.abs(x)))


def _gelu(x):
    return 0.5 * x * (1.0 + lax.erf(x * 0.7071067811865476))


def _ln(x, g, b):
    m = jnp.mean(x, axis=-1, keepdims=True)
    v = jnp.mean((x - m) * (x - m), axis=-1, keepdims=True)
    return (x - m) * lax.rsqrt(v + 1e-5) * g + b


def _matT(x, w):
    return lax.dot_general(x, w, (((1,), (1,)), ((), ())),
                           preferred_element_type=jnp.float32)


_BLK = 1000


def _prelude_body(x_ref, wr_ref, br_ref, w1_ref, b1_ref, w2_ref, b2_ref,
                  grb_ref, brb_ref, gn_ref, bn_ref,
                  rate_ref, gamma_ref, xres_ref):
    x = x_ref[...]
    xres_ref[...] = _ln(x, gn_ref[...], bn_ref[...])
    rate_ref[...] = _softplus(_matT(x, wr_ref[...]) + br_ref[...])
    t = _softplus(_matT(x, w1_ref[...]) + b1_ref[...])
    gamma_ref[...] = _ln(_matT(t, w2_ref[...]) + b2_ref[...],
                         grb_ref[...], brb_ref[...])


def _prelude_tc(x, wr, br, w1, b1, w2, b2, grb, brb, gn, bn):
    n = x.shape[0]
    grid = (n // _BLK,)
    row_spec = pl.BlockSpec((_BLK, D), lambda i: (i, 0))
    w_spec = pl.BlockSpec((D, D), lambda i: (0, 0))
    v_spec = pl.BlockSpec((1, D), lambda i: (0, 0))
    shp = jax.ShapeDtypeStruct((n, D), jnp.float32)
    return pl.pallas_call(
        _prelude_body,
        grid=grid,
        in_specs=[row_spec,
                  w_spec, v_spec, w_spec, v_spec, w_spec, v_spec,
                  v_spec, v_spec, v_spec, v_spec],
        out_specs=[row_spec, row_spec, row_spec],
        out_shape=[shp, shp, shp],
    )(x, wr, br, w1, b1, w2, b2, grb, brb, gn, bn)


def _post_body(rate_ref, gamma_ref, xres_ref, a0_ref, a1_ref, deg_ref,
               wf1_ref, bf1_ref, wf2_ref, bf2_ref, out_ref):
    rate = rate_ref[...]
    agg = a0_ref[...] + a1_ref[...]
    h = (rate * agg + gamma_ref[...]) / (1.0 + rate * deg_ref[...] + EPS)
    u = _gelu(_matT(h, wf1_ref[...]) + bf1_ref[...])
    out_ref[...] = _matT(u, wf2_ref[...]) + bf2_ref[...] + xres_ref[...]


def _post_tc(rate, gamma, xres, agg0, agg1, deg2d, wf1, bf1, wf2, bf2):
    n = rate.shape[0]
    grid = (n // _BLK,)
    row_spec = pl.BlockSpec((_BLK, D), lambda i: (i, 0))
    deg_spec = pl.BlockSpec((_BLK, 1), lambda i: (i, 0))
    w_spec = pl.BlockSpec((D, D), lambda i: (0, 0))
    v_spec = pl.BlockSpec((1, D), lambda i: (0, 0))
    return pl.pallas_call(
        _post_body,
        grid=grid,
        in_specs=[row_spec, row_spec, row_spec, row_spec, row_spec,
                  deg_spec, w_spec, v_spec, w_spec, v_spec],
        out_specs=row_spec,
        out_shape=jax.ShapeDtypeStruct((n, D), jnp.float32),
    )(rate, gamma, xres, agg0, agg1, deg2d, wf1, bf1, wf2, bf2)


def kernel(x, edge_index, degree, W_rate, b_rate, W_rb1, b_rb1, W_rb2, b_rb2,
           g_rb, beta_rb, W_fc1, b_fc1, W_fc2, b_fc2, g_norm, beta_norm):
    e = edge_index.shape[1]
    quantum = NW * CHUNK
    e_pad = ((e + quantum - 1) // quantum) * quantum
    pad = e_pad - e
    src_p = jnp.concatenate(
        [edge_index[0], jnp.zeros((pad,), jnp.int32)]) if pad else edge_index[0]
    dst_p = jnp.concatenate(
        [edge_index[1], jnp.full((pad,), N_NODES, jnp.int32)]) if pad else edge_index[1]
    n_chunks = e_pad // quantum
    src_r = src_p.reshape(n_chunks, NW, CHUNK).transpose(1, 0, 2)
    dst_r = dst_p.reshape(n_chunks, NW, CHUNK).transpose(1, 0, 2)
    idx_pairs = jnp.stack([src_r, dst_r], axis=2)
    zeros_hbm = jnp.zeros((ROWS_PER_TILE, D), jnp.float32)

    agg = _seg_sum_sc(x, idx_pairs, zeros_hbm)

    deg2d = degree[:, None]
    vec = lambda a: a.reshape(1, D)
    rate, gamma, xres = _prelude_tc(
        x, W_rate, vec(b_rate), W_rb1, vec(b_rb1), W_rb2, vec(b_rb2),
        vec(g_rb), vec(beta_rb), vec(g_norm), vec(beta_norm))
    out = _post_tc(rate, gamma, xres, agg[0, :N_NODES], agg[1, :N_NODES],
                   deg2d, W_fc1, vec(b_fc1), W_fc2, vec(b_fc2))
    return out


# speedup vs baseline: 1.8303x; 1.2454x over previous
import functools

import jax
import jax.numpy as jnp
from jax import lax
from jax.experimental import pallas as pl
from jax.experimental.pallas import tpu as pltpu
from jax.experimental.pallas import tpu_sc as plsc

EPS = 1e-4
N_NODES = 10000
D = 128

NC, NS = 2, 16
NW = NC * NS
CHUNK = 128
AGG_ROWS = 10240
ROWS_PER_TILE = AGG_ROWS // NS


def _seg_sum_sc(x, idx_pairs, zeros_hbm):
    n_chunks = idx_pairs.shape[1]
    mesh = plsc.VectorSubcoreMesh(core_axis_name="c", subcore_axis_name="s")

    @functools.partial(
        pl.kernel,
        out_type=jax.ShapeDtypeStruct((NC, AGG_ROWS, D), jnp.float32),
        mesh=mesh,
        scratch_types=[
            pltpu.VMEM((2, CHUNK), jnp.int32),
            pltpu.VMEM((CHUNK, D), jnp.float32),
            pltpu.VMEM_SHARED((AGG_ROWS, D), jnp.float32),
            pltpu.SemaphoreType.DMA,
        ],
    )
    def seg_kernel(x_hbm, idx_hbm, zero_hbm, out_hbm,
                   idx_v, rows_v, agg_sh, sem):
        c = lax.axis_index("c")
        s = lax.axis_index("s")
        wid = c * NS + s
        pltpu.sync_copy(zero_hbm,
                        agg_sh.at[pl.ds(s * ROWS_PER_TILE, ROWS_PER_TILE)])
        plsc.subcore_barrier()

        def body(j, carry):
            pltpu.sync_copy(idx_hbm.at[wid, j], idx_v)
            pltpu.async_copy(x_hbm.at[idx_v.at[0]], rows_v, sem).wait()
            pltpu.sync_copy(rows_v, agg_sh.at[idx_v.at[1]], add=True)
            return carry

        lax.fori_loop(0, n_chunks, body, 0)
        plsc.subcore_barrier()
        pltpu.sync_copy(agg_sh.at[pl.ds(s * ROWS_PER_TILE, ROWS_PER_TILE)],
                        out_hbm.at[c, pl.ds(s * ROWS_PER_TILE, ROWS_PER_TILE)])

    return seg_kernel(x, idx_pairs, zeros_hbm)


def _softplus(x):
    return jnp.maximum(x, 0.0) + jnp.log1p(jnp.exp(-jnp.abs(x)))


def _gelu(x):
    return 0.5 * x * (1.0 + lax.erf(x * 0.7071067811865476))


def _ln(x, g, b):
    m = jnp.mean(x, axis=-1, keepdims=True)
    v = jnp.mean((x - m) * (x - m), axis=-1, keepdims=True)
    return (x - m) * lax.rsqrt(v + 1e-5) * g + b


def _matT(x, w):
    return lax.dot_general(x, w, (((1,), (1,)), ((), ())),
                           preferred_element_type=jnp.float32)


_BLK = 1000


def _prelude_body(x_ref, wr_ref, br_ref, w1_ref, b1_ref, w2_ref, b2_ref,
                  grb_ref, brb_ref, gn_ref, bn_ref,
                  rate_ref, gamma_ref, xres_ref):
    x = x_ref[...]
    xres_ref[...] = _ln(x, gn_ref[...], bn_ref[...])
    rate_ref[...] = _softplus(_matT(x, wr_ref[...]) + br_ref[...])
    t = _softplus(_matT(x, w1_ref[...]) + b1_ref[...])
    gamma_ref[...] = _ln(_matT(t, w2_ref[...]) + b2_ref[...],
                         grb_ref[...], brb_ref[...])


def _prelude_tc(x, wr, br, w1, b1, w2, b2, grb, brb, gn, bn):
    n = x.shape[0]
    grid = (n // _BLK,)
    row_spec = pl.BlockSpec((_BLK, D), lambda i: (i, 0))
    w_spec = pl.BlockSpec((D, D), lambda i: (0, 0))
    v_spec = pl.BlockSpec((1, D), lambda i: (0, 0))
    shp = jax.ShapeDtypeStruct((n, D), jnp.float32)
    return pl.pallas_call(
        _prelude_body,
        grid=grid,
        in_specs=[row_spec,
                  w_spec, v_spec, w_spec, v_spec, w_spec, v_spec,
                  v_spec, v_spec, v_spec, v_spec],
        out_specs=[row_spec, row_spec, row_spec],
        out_shape=[shp, shp, shp],
    )(x, wr, br, w1, b1, w2, b2, grb, brb, gn, bn)


def _post_body(rate_ref, gamma_ref, xres_ref, a0_ref, a1_ref, deg_ref,
               wf1_ref, bf1_ref, wf2_ref, bf2_ref, out_ref):
    rate = rate_ref[...]
    agg = a0_ref[...] + a1_ref[...]
    h = (rate * agg + gamma_ref[...]) / (1.0 + rate * deg_ref[...] + EPS)
    u = _gelu(_matT(h, wf1_ref[...]) + bf1_ref[...])
    out_ref[...] = _matT(u, wf2_ref[...]) + bf2_ref[...] + xres_ref[...]


def _post_tc(rate, gamma, xres, agg0, agg1, deg2d, wf1, bf1, wf2, bf2):
    n = rate.shape[0]
    grid = (n // _BLK,)
    row_spec = pl.BlockSpec((_BLK, D), lambda i: (i, 0))
    deg_spec = pl.BlockSpec((_BLK, 1), lambda i: (i, 0))
    w_spec = pl.BlockSpec((D, D), lambda i: (0, 0))
    v_spec = pl.BlockSpec((1, D), lambda i: (0, 0))
    return pl.pallas_call(
        _post_body,
        grid=grid,
        in_specs=[row_spec, row_spec, row_spec, row_spec, row_spec,
                  deg_spec, w_spec, v_spec, w_spec, v_spec],
        out_specs=row_spec,
        out_shape=jax.ShapeDtypeStruct((n, D), jnp.float32),
    )(rate, gamma, xres, agg0, agg1, deg2d, wf1, bf1, wf2, bf2)


def kernel(x, edge_index, degree, W_rate, b_rate, W_rb1, b_rb1, W_rb2, b_rb2,
           g_rb, beta_rb, W_fc1, b_fc1, W_fc2, b_fc2, g_norm, beta_norm):
    e = edge_index.shape[1]
    quantum = NW * CHUNK
    e_pad = ((e + quantum - 1) // quantum) * quantum
    pad = e_pad - e
    src_p = jnp.concatenate(
        [edge_index[0], jnp.zeros((pad,), jnp.int32)]) if pad else edge_index[0]
    dst_p = jnp.concatenate(
        [edge_index[1], jnp.full((pad,), N_NODES, jnp.int32)]) if pad else edge_index[1]
    n_chunks = e_pad // quantum
    idx_pairs = jnp.stack(
        [src_p.reshape(NW, n_chunks, CHUNK),
         dst_p.reshape(NW, n_chunks, CHUNK)], axis=2)
    zeros_hbm = jnp.zeros((ROWS_PER_TILE, D), jnp.float32)

    agg = _seg_sum_sc(x, idx_pairs, zeros_hbm)

    deg2d = degree[:, None]
    vec = lambda a: a.reshape(1, D)
    rate, gamma, xres = _prelude_tc(
        x, W_rate, vec(b_rate), W_rb1, vec(b_rb1), W_rb2, vec(b_rb2),
        vec(g_rb), vec(beta_rb), vec(g_norm), vec(beta_norm))
    out = _post_tc(rate, gamma, xres, agg[0, :N_NODES], agg[1, :N_NODES],
                   deg2d, W_fc1, vec(b_fc1), W_fc2, vec(b_fc2))
    return out
